# baseline (speedup 1.0000x reference)
import functools

import jax
import jax.numpy as jnp
from jax import lax
from jax.experimental import pallas as pl
from jax.experimental.pallas import tpu as pltpu
from jax.experimental.pallas import tpu_sc as plsc

VOCAB = 100000
GOVS = 50
EMBED = 16
BATCH = 1024
CTX = 40

_IDX_CHUNK = 128
_VT = 2048


def _combine_sc(context, gov, word_emb, gov_emb):
    info = plsc.get_sparse_core_info()
    nc, ns = info.num_cores, info.num_subcores
    nw = nc * ns
    bpw = BATCH // nw
    ipw = bpw * CTX
    nch = ipw // _IDX_CHUNK
    ctx_flat = context.reshape(BATCH * CTX)

    mesh = plsc.VectorSubcoreMesh(core_axis_name="c", subcore_axis_name="s")

    @functools.partial(
        pl.kernel,
        out_type=jax.ShapeDtypeStruct((BATCH, EMBED), jnp.float32),
        mesh=mesh,
        scratch_types=[
            pltpu.VMEM((ipw,), jnp.int32),
            pltpu.VMEM((ipw, EMBED), jnp.float32),
            pltpu.VMEM((bpw,), jnp.int32),
            pltpu.VMEM((bpw, EMBED), jnp.float32),
            pltpu.VMEM((bpw, EMBED), jnp.float32),
            pltpu.SemaphoreType.DMA,
        ],
        compiler_params=pltpu.CompilerParams(use_tc_tiling_on_sc=False),
    )
    def combine(ctx_hbm, gov_hbm, wemb_hbm, gemb_hbm, out_hbm,
                idx_v, rows_v, gidx_v, grows_v, out_v, sem):
        wid = lax.axis_index("s") * nc + lax.axis_index("c")
        pltpu.sync_copy(ctx_hbm.at[pl.ds(wid * ipw, ipw)], idx_v)
        pltpu.sync_copy(gov_hbm.at[pl.ds(wid * bpw, bpw)], gidx_v)
        copies = [
            pltpu.async_copy(wemb_hbm.at[idx_v.at[pl.ds(k * _IDX_CHUNK,
                                                        _IDX_CHUNK)]],
                             rows_v.at[pl.ds(k * _IDX_CHUNK, _IDX_CHUNK)],
                             sem)
            for k in range(nch)
        ]
        copies.append(pltpu.async_copy(gemb_hbm.at[gidx_v], grows_v, sem))
        for c in copies:
            c.wait()

        def row_body(r, _):
            def acc_body(j, acc):
                return acc + rows_v[r * CTX + j, :]
            s = lax.fori_loop(0, CTX, acc_body,
                              jnp.zeros((EMBED,), jnp.float32))
            out_v[r, :] = s * (1.0 / CTX) + grows_v[r, :]
            return 0

        lax.fori_loop(0, bpw, row_body, 0)
        pltpu.sync_copy(out_v, out_hbm.at[pl.ds(wid * bpw, bpw)])

    return combine(ctx_flat, gov, word_emb, gov_emb)


def _project_tc(combined, W, b):
    nvt = pl.cdiv(VOCAB, _VT)
    b2 = b.reshape(1, VOCAB)

    def mm(comb_ref, w_ref, b_ref, out_ref):
        out_ref[...] = lax.dot_general(
            comb_ref[...], w_ref[...],
            dimension_numbers=(((1,), (1,)), ((), ())),
            preferred_element_type=jnp.float32,
        ) + b_ref[...]

    return pl.pallas_call(
        mm,
        grid=(nvt,),
        in_specs=[
            pl.BlockSpec((BATCH, EMBED), lambda i: (0, 0)),
            pl.BlockSpec((_VT, EMBED), lambda i: (i, 0)),
            pl.BlockSpec((1, _VT), lambda i: (0, i)),
        ],
        out_specs=pl.BlockSpec((BATCH, _VT), lambda i: (0, i)),
        out_shape=jax.ShapeDtypeStruct((BATCH, VOCAB), jnp.float32),
    )(combined, W, b2)


def kernel(context, gov, word_emb, gov_emb, W, b):
    combined = _combine_sc(context, gov, word_emb, gov_emb)
    return _project_tc(combined, W, b)


# speedup vs baseline: 2.9133x; 2.9133x over previous
import functools

import jax
import jax.numpy as jnp
from jax import lax
from jax.experimental import pallas as pl
from jax.experimental.pallas import tpu as pltpu
from jax.experimental.pallas import tpu_sc as plsc

VOCAB = 100000
GOVS = 50
EMBED = 16
BATCH = 1024
CTX = 40

_IDX_CHUNK = 128
_VT = 2048


def _combine_sc(context, gov, word_emb, gov_emb):
    info = plsc.get_sparse_core_info()
    nc, ns = info.num_cores, info.num_subcores
    nw = nc * ns
    bpw = BATCH // nw
    ipw = bpw * CTX
    nch = ipw // _IDX_CHUNK
    ctx_flat = context.reshape(BATCH * CTX)

    mesh = plsc.VectorSubcoreMesh(core_axis_name="c", subcore_axis_name="s")

    @functools.partial(
        pl.kernel,
        out_type=jax.ShapeDtypeStruct((BATCH, EMBED), jnp.float32),
        mesh=mesh,
        scratch_types=[
            pltpu.VMEM((ipw,), jnp.int32),
            pltpu.VMEM((ipw, EMBED), jnp.float32),
            pltpu.VMEM((bpw,), jnp.int32),
            pltpu.VMEM((bpw, EMBED), jnp.float32),
            pltpu.VMEM((bpw, EMBED), jnp.float32),
            pltpu.SemaphoreType.DMA,
        ],
        compiler_params=pltpu.CompilerParams(use_tc_tiling_on_sc=False),
    )
    def combine(ctx_hbm, gov_hbm, wemb_hbm, gemb_hbm, out_hbm,
                idx_v, rows_v, gidx_v, grows_v, out_v, sem):
        wid = lax.axis_index("s") * nc + lax.axis_index("c")
        pltpu.sync_copy(ctx_hbm.at[pl.ds(wid * ipw, ipw)], idx_v)
        pltpu.sync_copy(gov_hbm.at[pl.ds(wid * bpw, bpw)], gidx_v)
        copies = [
            pltpu.async_copy(wemb_hbm.at[idx_v.at[pl.ds(k * _IDX_CHUNK,
                                                        _IDX_CHUNK)]],
                             rows_v.at[pl.ds(k * _IDX_CHUNK, _IDX_CHUNK)],
                             sem)
            for k in range(nch)
        ]
        copies.append(pltpu.async_copy(gemb_hbm.at[gidx_v], grows_v, sem))
        for c in copies:
            c.wait()

        def row_body(r, _):
            def acc_body(j, acc):
                return acc + rows_v[r * CTX + j, :]
            s = lax.fori_loop(0, CTX, acc_body,
                              jnp.zeros((EMBED,), jnp.float32))
            out_v[r, :] = s * (1.0 / CTX) + grows_v[r, :]
            return 0

        lax.fori_loop(0, bpw, row_body, 0)
        pltpu.sync_copy(out_v, out_hbm.at[pl.ds(wid * bpw, bpw)])

    return combine(ctx_flat, gov, word_emb, gov_emb)


def _project_tc_t(comb_aug, w_aug_t):
    nvt = pl.cdiv(VOCAB, _VT)
    ka = comb_aug.shape[1]

    def mm(w_ref, comb_ref, out_ref):
        out_ref[...] = lax.dot_general(
            w_ref[...], comb_ref[...],
            dimension_numbers=(((0,), (1,)), ((), ())),
            preferred_element_type=jnp.float32,
        )

    return pl.pallas_call(
        mm,
        grid=(nvt,),
        in_specs=[
            pl.BlockSpec((ka, _VT), lambda i: (0, i)),
            pl.BlockSpec((BATCH, ka), lambda i: (0, 0)),
        ],
        out_specs=pl.BlockSpec((_VT, BATCH), lambda i: (i, 0)),
        out_shape=jax.ShapeDtypeStruct((VOCAB, BATCH), jnp.float32),
    )(w_aug_t, comb_aug)


def kernel(context, gov, word_emb, gov_emb, W, b):
    combined = _combine_sc(context, gov, word_emb, gov_emb)
    comb_aug = jnp.concatenate(
        [combined, jnp.ones((BATCH, 1), jnp.float32)], axis=1)
    w_aug_t = jnp.concatenate([W.T, b[None, :]], axis=0)
    return _project_tc_t(comb_aug, w_aug_t).T
